# Initial kernel scaffold; baseline (speedup 1.0000x reference)
import jax
import jax.numpy as jnp
from jax.experimental import pallas as pl


def kernel(x, edge_index, W1, b1, W2, b2, W3, b3):
    raise NotImplementedError("write your pallas kernel here")


# speedup vs baseline: 27.6133x; 27.6133x over previous
import functools

import jax
import jax.numpy as jnp
from jax import lax
from jax.experimental import pallas as pl
from jax.experimental.pallas import tpu as pltpu
from jax.experimental.pallas import tpu_sc as plsc

N = 10000
E = 320000
NC, NS, L = 2, 16, 16
NW = NC * NS

NPAD = 10240
ROWS_PER_TILE = NPAD // NS

DEG_CHUNK = 128
DEG_CHUNKS = 157
DEG_PER_TILE = DEG_CHUNKS * DEG_CHUNK
DEG_TOTAL = NS * DEG_PER_TILE

AGG_CHUNK = 80
AGG_CHUNKS = 125
EDGES_PER_TILE = AGG_CHUNKS * AGG_CHUNK


def _sc_mesh():
    return plsc.VectorSubcoreMesh(
        core_axis_name="c", subcore_axis_name="s",
        num_cores=NC, num_subcores=NS)


@functools.cache
def _make_deg():
    return functools.partial(
        pl.kernel,
        out_type=jax.ShapeDtypeStruct((NPAD,), jnp.float32),
        mesh=_sc_mesh(),
        compiler_params=pltpu.CompilerParams(use_tc_tiling_on_sc=False),
        scratch_types=[
            pltpu.VMEM((DEG_CHUNKS, DEG_CHUNK), jnp.int32),
            pltpu.VMEM((DEG_CHUNK,), jnp.float32),
            pltpu.VMEM_SHARED((NPAD,), jnp.float32),
        ],
    )(_deg_body)


def _deg_body(dst_hbm, zeros_hbm, deg_out, dstv, onesv, acc):
    c = lax.axis_index("c")
    s = lax.axis_index("s")
    one = jnp.full((L,), 1.0, dtype=jnp.float32)
    for i in range(DEG_CHUNK // L):
        onesv[pl.ds(i * L, L)] = one
    pltpu.sync_copy(dst_hbm.at[s], dstv)
    stripe = pl.ds(s * (NPAD // NS), NPAD // NS)
    pltpu.sync_copy(zeros_hbm.at[stripe], acc.at[stripe])
    plsc.subcore_barrier()

    def body(j, carry):
        pltpu.sync_copy(onesv, acc.at[dstv.at[j]], add=True)
        return carry

    lax.fori_loop(0, DEG_CHUNKS, body, 0)
    plsc.subcore_barrier()

    @pl.when(c == 0)
    def _():
        pltpu.sync_copy(acc.at[stripe], deg_out.at[stripe])


@functools.cache
def _make_agg(F):
    @functools.partial(
        pl.kernel,
        out_type=jax.ShapeDtypeStruct((NC, NPAD, F), jnp.float32),
        mesh=_sc_mesh(),
        compiler_params=pltpu.CompilerParams(use_tc_tiling_on_sc=False),
        scratch_types=[
            pltpu.VMEM((AGG_CHUNKS, AGG_CHUNK), jnp.int32),
            pltpu.VMEM((AGG_CHUNKS, AGG_CHUNK), jnp.int32),
            pltpu.VMEM((AGG_CHUNK, F), jnp.float32),
            pltpu.VMEM((AGG_CHUNK, F), jnp.float32),
            pltpu.VMEM_SHARED((NPAD, F), jnp.float32),
            pltpu.SemaphoreType.DMA,
            pltpu.SemaphoreType.DMA,
        ],
    )
    def agg(src_hbm, dst_hbm, g_hbm, zeros_hbm, out_hbm,
            srcv, dstv, rows0, rows1, acc, sem0, sem1):
        c = lax.axis_index("c")
        s = lax.axis_index("s")
        w = c * NS + s
        pltpu.sync_copy(src_hbm.at[w], srcv)
        pltpu.sync_copy(dst_hbm.at[w], dstv)
        stripe = pl.ds(s * ROWS_PER_TILE, ROWS_PER_TILE)
        pltpu.sync_copy(zeros_hbm.at[stripe], acc.at[stripe])
        plsc.subcore_barrier()

        pltpu.async_copy(g_hbm.at[srcv.at[0]], rows0, sem0)

        def pair(k, carry):
            j0 = 2 * k
            j1 = j0 + 1
            j2 = j0 + 2
            pltpu.make_async_copy(g_hbm.at[srcv.at[j0]], rows0, sem0).wait()
            pltpu.async_copy(g_hbm.at[srcv.at[j1]], rows1, sem1)
            pltpu.sync_copy(rows0, acc.at[dstv.at[j0]], add=True)
            pltpu.make_async_copy(g_hbm.at[srcv.at[j1]], rows1, sem1).wait()

            @pl.when(j2 < AGG_CHUNKS)
            def _():
                pltpu.async_copy(g_hbm.at[srcv.at[j2]], rows0, sem0)

            pltpu.sync_copy(rows1, acc.at[dstv.at[j1]], add=True)
            return carry

        lax.fori_loop(0, AGG_CHUNKS // 2, pair, 0)
        last = AGG_CHUNKS - 1
        pltpu.make_async_copy(g_hbm.at[srcv.at[last]], rows0, sem0).wait()
        pltpu.sync_copy(rows0, acc.at[dstv.at[last]], add=True)
        plsc.subcore_barrier()
        pltpu.sync_copy(acc.at[stripe], out_hbm.at[c].at[stripe])

    return agg


RB = 2000
GRID = N // RB


def _dinv(deg_blk):
    return lax.rsqrt(deg_blk + 1.0)


def _stage1_body(x_ref, w_ref, deg_ref, o_ref):
    dinv = _dinv(deg_ref[...])
    h = jnp.dot(x_ref[...], w_ref[...], preferred_element_type=jnp.float32)
    o_ref[...] = h * dinv


def _stage1(x, W1, deg):
    return pl.pallas_call(
        _stage1_body,
        out_shape=jax.ShapeDtypeStruct((N, 16), jnp.float32),
        grid=(GRID,),
        in_specs=[
            pl.BlockSpec((RB, 128), lambda i: (i, 0)),
            pl.BlockSpec((128, 16), lambda i: (0, 0)),
            pl.BlockSpec((RB, 1), lambda i: (i, 0)),
        ],
        out_specs=pl.BlockSpec((RB, 16), lambda i: (i, 0)),
    )(x, W1, deg)


def _mid_body(p_ref, g_ref, deg_ref, b_ref, w_ref, o_ref):
    dinv = _dinv(deg_ref[...])
    z = (p_ref[0] + p_ref[1] + g_ref[...]) * dinv + b_ref[...]
    h = jnp.maximum(z, 0.0)
    o_ref[...] = jnp.dot(h, w_ref[...], preferred_element_type=jnp.float32) * dinv


def _mid(p, g, deg, b, W):
    F = g.shape[1]
    F2 = W.shape[1]
    return pl.pallas_call(
        _mid_body,
        out_shape=jax.ShapeDtypeStruct((N, F2), jnp.float32),
        grid=(GRID,),
        in_specs=[
            pl.BlockSpec((2, RB, F), lambda i: (0, i, 0)),
            pl.BlockSpec((RB, F), lambda i: (i, 0)),
            pl.BlockSpec((RB, 1), lambda i: (i, 0)),
            pl.BlockSpec((1, F), lambda i: (0, 0)),
            pl.BlockSpec((F, F2), lambda i: (0, 0)),
        ],
        out_specs=pl.BlockSpec((RB, F2), lambda i: (i, 0)),
    )(p, g, deg, b, W)


def _final_body(p_ref, g_ref, deg_ref, b_ref, o_ref):
    dinv = _dinv(deg_ref[...])
    z = (p_ref[0] + p_ref[1] + g_ref[...]) * dinv + b_ref[...]
    z2 = z[:, 0:2]
    m = jnp.max(z2, axis=1, keepdims=True)
    e = jnp.exp(z2 - m)
    lse = jnp.log(e[:, 0:1] + e[:, 1:2]) + m
    o_ref[...] = z2 - lse


def _final(p, g, deg, b):
    return pl.pallas_call(
        _final_body,
        out_shape=jax.ShapeDtypeStruct((N, 2), jnp.float32),
        grid=(GRID,),
        in_specs=[
            pl.BlockSpec((2, RB, 8), lambda i: (0, i, 0)),
            pl.BlockSpec((RB, 8), lambda i: (i, 0)),
            pl.BlockSpec((RB, 1), lambda i: (i, 0)),
            pl.BlockSpec((1, 8), lambda i: (0, 0)),
        ],
        out_specs=pl.BlockSpec((RB, 2), lambda i: (i, 0)),
    )(p, g, deg, b)


def kernel(x, edge_index, W1, b1, W2, b2, W3, b3):
    src = edge_index[0]
    dst = edge_index[1]
    src3 = src.reshape(NW, AGG_CHUNKS, AGG_CHUNK)
    dst3 = dst.reshape(NW, AGG_CHUNKS, AGG_CHUNK)
    padi = N + (jnp.arange(DEG_TOTAL - E, dtype=jnp.int32) % (NPAD - N))
    dstp = jnp.concatenate([dst, padi]).reshape(NS, DEG_CHUNKS, DEG_CHUNK)

    deg = _make_deg()(dstp, jnp.zeros((NPAD,), jnp.float32))
    degc = deg.reshape(NPAD, 1)

    zeros16 = jnp.zeros((NPAD, 16), jnp.float32)
    zeros32 = jnp.zeros((NPAD, 32), jnp.float32)
    zeros8 = jnp.zeros((NPAD, 8), jnp.float32)

    g1 = _stage1(x, W1, degc)
    p1 = _make_agg(16)(src3, dst3, g1, zeros16)
    g2 = _mid(p1, g1, degc, b1.reshape(1, -1), W2)
    p2 = _make_agg(32)(src3, dst3, g2, zeros32)
    W3p = jnp.pad(W3, ((0, 0), (0, 6)))
    b3p = jnp.pad(b3, (0, 6)).reshape(1, 8)
    g3 = _mid(p2, g2, degc, b2.reshape(1, -1), W3p)
    p3 = _make_agg(8)(src3, dst3, g3, zeros8)
    return _final(p3, g3, degc, b3p)


# speedup vs baseline: 46.4413x; 1.6818x over previous
import functools

import jax
import jax.numpy as jnp
from jax import lax
from jax.experimental import pallas as pl
from jax.experimental.pallas import tpu as pltpu
from jax.experimental.pallas import tpu_sc as plsc

N = 10000
E = 320000
NC, NS, L = 2, 16, 16
NW = NC * NS

NPAD = 10240
ROWS_PER_TILE = NPAD // NS

DEG_CHUNK = 80
DEG_CHUNKS = 250

AGG_CHUNK = 80
AGG_CHUNKS = 125
EDGES_PER_TILE = AGG_CHUNKS * AGG_CHUNK


def _sc_mesh():
    return plsc.VectorSubcoreMesh(
        core_axis_name="c", subcore_axis_name="s",
        num_cores=NC, num_subcores=NS)


@functools.cache
def _make_deg():
    return functools.partial(
        pl.kernel,
        out_type=jax.ShapeDtypeStruct((NPAD,), jnp.float32),
        mesh=_sc_mesh(),
        compiler_params=pltpu.CompilerParams(use_tc_tiling_on_sc=False),
        scratch_types=[
            pltpu.VMEM((DEG_CHUNKS, DEG_CHUNK), jnp.int32),
            pltpu.VMEM((DEG_CHUNK,), jnp.float32),
            pltpu.VMEM_SHARED((NPAD,), jnp.float32),
            pltpu.SemaphoreType.DMA,
            pltpu.SemaphoreType.DMA,
            pltpu.SemaphoreType.DMA,
            pltpu.SemaphoreType.DMA,
        ],
    )(_deg_body)


def _deg_body(dst_hbm, zeros_hbm, deg_out, dstv, onesv, acc, s0, s1, s2, s3):
    c = lax.axis_index("c")
    s = lax.axis_index("s")
    ssem = (s0, s1, s2, s3)
    one = jnp.full((L,), 1.0, dtype=jnp.float32)
    for i in range(DEG_CHUNK // L):
        onesv[pl.ds(i * L, L)] = one
    pltpu.sync_copy(dst_hbm.at[s], dstv)
    stripe = pl.ds(s * (NPAD // NS), NPAD // NS)
    pltpu.sync_copy(zeros_hbm.at[stripe], acc.at[stripe])
    plsc.subcore_barrier()

    def s_fire(j, slot):
        pltpu.async_copy(onesv, acc.at[dstv.at[j]], ssem[slot], add=True)

    def s_wait(j, slot):
        pltpu.make_async_copy(onesv, acc.at[dstv.at[j]], ssem[slot]).wait()

    def grp(k, carry):
        for b in range(4):
            j = 4 * k + b

            @pl.when(j >= 4)
            def _():
                s_wait(j - 4, b)

            s_fire(j, b)
        return carry

    lax.fori_loop(0, DEG_CHUNKS // 4, grp, 0)
    s_wait(DEG_CHUNKS - 6, 0)
    s_fire(DEG_CHUNKS - 2, 0)
    s_wait(DEG_CHUNKS - 5, 1)
    s_fire(DEG_CHUNKS - 1, 1)
    s_wait(DEG_CHUNKS - 4, 2)
    s_wait(DEG_CHUNKS - 3, 3)
    s_wait(DEG_CHUNKS - 2, 0)
    s_wait(DEG_CHUNKS - 1, 1)
    plsc.subcore_barrier()

    @pl.when(c == 0)
    def _():
        pltpu.sync_copy(acc.at[stripe], deg_out.at[stripe])


@functools.cache
def _make_agg(F):
    @functools.partial(
        pl.kernel,
        out_type=jax.ShapeDtypeStruct((NC, NPAD, F), jnp.float32),
        mesh=_sc_mesh(),
        compiler_params=pltpu.CompilerParams(use_tc_tiling_on_sc=False),
        scratch_types=[
            pltpu.VMEM((AGG_CHUNKS, AGG_CHUNK), jnp.int32),
            pltpu.VMEM((AGG_CHUNKS, AGG_CHUNK), jnp.int32),
            pltpu.VMEM((AGG_CHUNK, F), jnp.float32),
            pltpu.VMEM((AGG_CHUNK, F), jnp.float32),
            pltpu.VMEM((AGG_CHUNK, F), jnp.float32),
            pltpu.VMEM((AGG_CHUNK, F), jnp.float32),
            pltpu.VMEM_SHARED((NPAD, F), jnp.float32),
            pltpu.SemaphoreType.DMA,
            pltpu.SemaphoreType.DMA,
            pltpu.SemaphoreType.DMA,
            pltpu.SemaphoreType.DMA,
            pltpu.SemaphoreType.DMA,
            pltpu.SemaphoreType.DMA,
            pltpu.SemaphoreType.DMA,
            pltpu.SemaphoreType.DMA,
        ],
    )
    def agg(src_hbm, dst_hbm, g_hbm, zeros_hbm, out_hbm,
            srcv, dstv, r0, r1, r2, r3, acc,
            g0, g1, g2, g3, s0, s1, s2, s3):
        c = lax.axis_index("c")
        s = lax.axis_index("s")
        w = c * NS + s
        rows = (r0, r1, r2, r3)
        gsem = (g0, g1, g2, g3)
        ssem = (s0, s1, s2, s3)
        pltpu.sync_copy(src_hbm.at[w], srcv)
        pltpu.sync_copy(dst_hbm.at[w], dstv)
        stripe = pl.ds(s * ROWS_PER_TILE, ROWS_PER_TILE)
        pltpu.sync_copy(zeros_hbm.at[stripe], acc.at[stripe])
        plsc.subcore_barrier()

        def g_issue(j, slot):
            pltpu.async_copy(g_hbm.at[srcv.at[j]], rows[slot], gsem[slot])

        def g_wait(j, slot):
            pltpu.make_async_copy(g_hbm.at[srcv.at[j]], rows[slot], gsem[slot]).wait()

        def s_fire(j, slot):
            pltpu.async_copy(rows[slot], acc.at[dstv.at[j]], ssem[slot], add=True)

        def s_wait(j, slot):
            pltpu.make_async_copy(rows[slot], acc.at[dstv.at[j]], ssem[slot]).wait()

        g_issue(0, 0)
        g_issue(1, 1)

        def grp(k, carry):
            for b in range(4):
                j = 4 * k + b
                s2 = (b + 2) % 4

                @pl.when(j >= 2)
                def _():
                    s_wait(j - 2, s2)

                @pl.when(j + 2 < AGG_CHUNKS)
                def _():
                    g_issue(j + 2, s2)

                g_wait(j, b)
                s_fire(j, b)
            return carry

        lax.fori_loop(0, AGG_CHUNKS // 4, grp, 0)
        last = AGG_CHUNKS - 1
        s_wait(last - 2, 2)
        g_wait(last, 0)
        s_fire(last, 0)
        s_wait(last - 1, 3)
        s_wait(last, 0)
        plsc.subcore_barrier()
        pltpu.sync_copy(acc.at[stripe], out_hbm.at[c].at[stripe])

    return agg


RB = 2000
GRID = N // RB


def _dinv(deg_blk):
    return lax.rsqrt(deg_blk + 1.0)


def _mm_body(x_ref, w_ref, o_ref):
    o_ref[...] = jnp.dot(x_ref[...], w_ref[...],
                         preferred_element_type=jnp.float32)


def _mm(x, W1):
    return pl.pallas_call(
        _mm_body,
        out_shape=jax.ShapeDtypeStruct((N, 16), jnp.float32),
        grid=(GRID,),
        in_specs=[
            pl.BlockSpec((RB, 128), lambda i: (i, 0)),
            pl.BlockSpec((128, 16), lambda i: (0, 0)),
        ],
        out_specs=pl.BlockSpec((RB, 16), lambda i: (i, 0)),
    )(x, W1)


def _scale_body(h_ref, deg_ref, o_ref):
    o_ref[...] = h_ref[...] * _dinv(deg_ref[...])


def _scale(h, deg):
    return pl.pallas_call(
        _scale_body,
        out_shape=jax.ShapeDtypeStruct((N, 16), jnp.float32),
        grid=(GRID,),
        in_specs=[
            pl.BlockSpec((RB, 16), lambda i: (i, 0)),
            pl.BlockSpec((RB, 1), lambda i: (i, 0)),
        ],
        out_specs=pl.BlockSpec((RB, 16), lambda i: (i, 0)),
    )(h, deg)


def _mid_body(p_ref, g_ref, deg_ref, b_ref, w_ref, o_ref):
    dinv = _dinv(deg_ref[...])
    z = (p_ref[0] + p_ref[1] + g_ref[...]) * dinv + b_ref[...]
    h = jnp.maximum(z, 0.0)
    o_ref[...] = jnp.dot(h, w_ref[...], preferred_element_type=jnp.float32) * dinv


def _mid(p, g, deg, b, W):
    F = g.shape[1]
    F2 = W.shape[1]
    return pl.pallas_call(
        _mid_body,
        out_shape=jax.ShapeDtypeStruct((N, F2), jnp.float32),
        grid=(GRID,),
        in_specs=[
            pl.BlockSpec((2, RB, F), lambda i: (0, i, 0)),
            pl.BlockSpec((RB, F), lambda i: (i, 0)),
            pl.BlockSpec((RB, 1), lambda i: (i, 0)),
            pl.BlockSpec((1, F), lambda i: (0, 0)),
            pl.BlockSpec((F, F2), lambda i: (0, 0)),
        ],
        out_specs=pl.BlockSpec((RB, F2), lambda i: (i, 0)),
    )(p, g, deg, b, W)


def _final_body(p_ref, g_ref, deg_ref, b_ref, o_ref):
    dinv = _dinv(deg_ref[...])
    z = (p_ref[0] + p_ref[1] + g_ref[...]) * dinv + b_ref[...]
    z2 = z[:, 0:2]
    m = jnp.max(z2, axis=1, keepdims=True)
    e = jnp.exp(z2 - m)
    lse = jnp.log(e[:, 0:1] + e[:, 1:2]) + m
    o_ref[...] = z2 - lse


def _final(p, g, deg, b):
    return pl.pallas_call(
        _final_body,
        out_shape=jax.ShapeDtypeStruct((N, 2), jnp.float32),
        grid=(GRID,),
        in_specs=[
            pl.BlockSpec((2, RB, 8), lambda i: (0, i, 0)),
            pl.BlockSpec((RB, 8), lambda i: (i, 0)),
            pl.BlockSpec((RB, 1), lambda i: (i, 0)),
            pl.BlockSpec((1, 8), lambda i: (0, 0)),
        ],
        out_specs=pl.BlockSpec((RB, 2), lambda i: (i, 0)),
    )(p, g, deg, b)


def kernel(x, edge_index, W1, b1, W2, b2, W3, b3):
    src = edge_index[0]
    dst = edge_index[1]
    src3 = src.reshape(NW, AGG_CHUNKS, AGG_CHUNK)
    dst3 = dst.reshape(NW, AGG_CHUNKS, AGG_CHUNK)
    dstp = dst.reshape(NS, DEG_CHUNKS, DEG_CHUNK)

    h1 = _mm(x, W1)
    deg = _make_deg()(dstp, jnp.zeros((NPAD,), jnp.float32))
    degc = deg.reshape(NPAD, 1)

    zeros16 = jnp.zeros((NPAD, 16), jnp.float32)
    zeros32 = jnp.zeros((NPAD, 32), jnp.float32)
    zeros8 = jnp.zeros((NPAD, 8), jnp.float32)

    g1 = _scale(h1, degc)
    p1 = _make_agg(16)(src3, dst3, g1, zeros16)
    g2 = _mid(p1, g1, degc, b1.reshape(1, -1), W2)
    p2 = _make_agg(32)(src3, dst3, g2, zeros32)
    W3p = jnp.pad(W3, ((0, 0), (0, 6)))
    b3p = jnp.pad(b3, (0, 6)).reshape(1, 8)
    g3 = _mid(p2, g2, degc, b2.reshape(1, -1), W3p)
    p3 = _make_agg(8)(src3, dst3, g3, zeros8)
    return _final(p3, g3, degc, b3p)


# speedup vs baseline: 51.6805x; 1.1128x over previous
import functools

import jax
import jax.numpy as jnp
from jax import lax
from jax.experimental import pallas as pl
from jax.experimental.pallas import tpu as pltpu
from jax.experimental.pallas import tpu_sc as plsc

N = 10000
E = 320000
NC, NS, L = 2, 16, 16
NW = NC * NS

NPAD = 10240
ROWS_PER_TILE = NPAD // NS

CHUNK = 128
NCHUNKS = E // CHUNK
AGG_Q, AGG_R = NCHUNKS // NW, NCHUNKS % NW
AGG_MAX = AGG_Q + 1
DEG_Q, DEG_R = NCHUNKS // NS, NCHUNKS % NS
DEG_MAX = DEG_Q + 1


def _sc_mesh():
    return plsc.VectorSubcoreMesh(
        core_axis_name="c", subcore_axis_name="s",
        num_cores=NC, num_subcores=NS)


@functools.cache
def _make_deg():
    return functools.partial(
        pl.kernel,
        out_type=jax.ShapeDtypeStruct((NPAD,), jnp.float32),
        mesh=_sc_mesh(),
        compiler_params=pltpu.CompilerParams(use_tc_tiling_on_sc=False),
        scratch_types=[
            pltpu.VMEM((DEG_MAX, CHUNK), jnp.int32),
            pltpu.VMEM((CHUNK,), jnp.float32),
            pltpu.VMEM_SHARED((NPAD,), jnp.float32),
            pltpu.SemaphoreType.DMA,
            pltpu.SemaphoreType.DMA,
            pltpu.SemaphoreType.DMA,
            pltpu.SemaphoreType.DMA,
        ],
    )(_deg_body)


def _deg_body(dst_hbm, zeros_hbm, deg_out, dstv, onesv, acc, s0, s1, s2, s3):
    c = lax.axis_index("c")
    s = lax.axis_index("s")
    ssem = (s0, s1, s2, s3)
    nch = jnp.where(s >= NS - DEG_R, DEG_Q + 1, DEG_Q)
    base = DEG_Q * s + jnp.maximum(s - (NS - DEG_R), 0)
    one = jnp.full((L,), 1.0, dtype=jnp.float32)
    for i in range(CHUNK // L):
        onesv[pl.ds(i * L, L)] = one
    pltpu.sync_copy(dst_hbm.at[pl.ds(base, DEG_MAX)], dstv)
    stripe = pl.ds(s * (NPAD // NS), NPAD // NS)
    pltpu.sync_copy(zeros_hbm.at[stripe], acc.at[stripe])
    plsc.subcore_barrier()

    def s_fire(j, slot):
        pltpu.async_copy(onesv, acc.at[dstv.at[j]], ssem[slot], add=True)

    def s_wait(j, slot):
        pltpu.make_async_copy(onesv, acc.at[dstv.at[j]], ssem[slot]).wait()

    def grp(k, carry):
        for b in range(4):
            j = 4 * k + b

            @pl.when((j >= 4) & (j - 4 < nch))
            def _():
                s_wait(j - 4, b)

            @pl.when(j < nch)
            def _():
                s_fire(j, b)
        return carry

    lax.fori_loop(0, (DEG_MAX + 4 + 3) // 4, grp, 0)
    plsc.subcore_barrier()

    @pl.when(c == 0)
    def _():
        pltpu.sync_copy(acc.at[stripe], deg_out.at[stripe])


@functools.cache
def _make_agg(F):
    @functools.partial(
        pl.kernel,
        out_type=jax.ShapeDtypeStruct((NC, NPAD, F), jnp.float32),
        mesh=_sc_mesh(),
        compiler_params=pltpu.CompilerParams(use_tc_tiling_on_sc=False),
        scratch_types=[
            pltpu.VMEM((AGG_MAX, CHUNK), jnp.int32),
            pltpu.VMEM((AGG_MAX, CHUNK), jnp.int32),
            pltpu.VMEM((CHUNK, F), jnp.float32),
            pltpu.VMEM((CHUNK, F), jnp.float32),
            pltpu.VMEM((CHUNK, F), jnp.float32),
            pltpu.VMEM((CHUNK, F), jnp.float32),
            pltpu.VMEM_SHARED((NPAD, F), jnp.float32),
            pltpu.SemaphoreType.DMA,
            pltpu.SemaphoreType.DMA,
            pltpu.SemaphoreType.DMA,
            pltpu.SemaphoreType.DMA,
            pltpu.SemaphoreType.DMA,
            pltpu.SemaphoreType.DMA,
            pltpu.SemaphoreType.DMA,
            pltpu.SemaphoreType.DMA,
        ],
    )
    def agg(src_hbm, dst_hbm, g_hbm, zeros_hbm, out_hbm,
            srcv, dstv, r0, r1, r2, r3, acc,
            g0, g1, g2, g3, s0, s1, s2, s3):
        c = lax.axis_index("c")
        s = lax.axis_index("s")
        w = c * NS + s
        rows = (r0, r1, r2, r3)
        gsem = (g0, g1, g2, g3)
        ssem = (s0, s1, s2, s3)
        nch = jnp.where(w >= NW - AGG_R, AGG_Q + 1, AGG_Q)
        base = AGG_Q * w + jnp.maximum(w - (NW - AGG_R), 0)
        pltpu.sync_copy(src_hbm.at[pl.ds(base, AGG_MAX)], srcv)
        pltpu.sync_copy(dst_hbm.at[pl.ds(base, AGG_MAX)], dstv)
        stripe = pl.ds(s * ROWS_PER_TILE, ROWS_PER_TILE)
        pltpu.sync_copy(zeros_hbm.at[stripe], acc.at[stripe])
        plsc.subcore_barrier()

        def g_issue(j, slot):
            pltpu.async_copy(g_hbm.at[srcv.at[j]], rows[slot], gsem[slot])

        def g_wait(j, slot):
            pltpu.make_async_copy(g_hbm.at[srcv.at[j]], rows[slot], gsem[slot]).wait()

        def s_fire(j, slot):
            pltpu.async_copy(rows[slot], acc.at[dstv.at[j]], ssem[slot], add=True)

        def s_wait(j, slot):
            pltpu.make_async_copy(rows[slot], acc.at[dstv.at[j]], ssem[slot]).wait()

        g_issue(0, 0)
        g_issue(1, 1)

        def grp(k, carry):
            for b in range(4):
                j = 4 * k + b
                s2 = (b + 2) % 4

                @pl.when((j >= 2) & (j - 2 < nch))
                def _():
                    s_wait(j - 2, s2)

                @pl.when(j + 2 < nch)
                def _():
                    g_issue(j + 2, s2)

                @pl.when(j < nch)
                def _():
                    g_wait(j, b)
                    s_fire(j, b)
            return carry

        lax.fori_loop(0, (AGG_MAX + 2 + 3) // 4, grp, 0)
        plsc.subcore_barrier()
        pltpu.sync_copy(acc.at[stripe], out_hbm.at[c].at[stripe])

    return agg


RB = 2000
GRID = N // RB


def _dinv(deg_blk):
    return lax.rsqrt(deg_blk + 1.0)


def _stage1_body(x_ref, w_ref, deg_ref, o_ref):
    h = jnp.dot(x_ref[...], w_ref[...], preferred_element_type=jnp.float32)
    o_ref[...] = h * _dinv(deg_ref[...])


def _stage1(x, W1, deg):
    return pl.pallas_call(
        _stage1_body,
        out_shape=jax.ShapeDtypeStruct((N, 16), jnp.float32),
        grid=(GRID,),
        in_specs=[
            pl.BlockSpec((RB, 128), lambda i: (i, 0)),
            pl.BlockSpec((128, 16), lambda i: (0, 0)),
            pl.BlockSpec((RB, 1), lambda i: (i, 0)),
        ],
        out_specs=pl.BlockSpec((RB, 16), lambda i: (i, 0)),
    )(x, W1, deg)


def _mid_body(p_ref, g_ref, deg_ref, b_ref, w_ref, o_ref):
    dinv = _dinv(deg_ref[...])
    z = (p_ref[0] + p_ref[1] + g_ref[...]) * dinv + b_ref[...]
    h = jnp.maximum(z, 0.0)
    o_ref[...] = jnp.dot(h, w_ref[...], preferred_element_type=jnp.float32) * dinv


def _mid(p, g, deg, b, W):
    F = g.shape[1]
    F2 = W.shape[1]
    return pl.pallas_call(
        _mid_body,
        out_shape=jax.ShapeDtypeStruct((N, F2), jnp.float32),
        grid=(GRID,),
        in_specs=[
            pl.BlockSpec((2, RB, F), lambda i: (0, i, 0)),
            pl.BlockSpec((RB, F), lambda i: (i, 0)),
            pl.BlockSpec((RB, 1), lambda i: (i, 0)),
            pl.BlockSpec((1, F), lambda i: (0, 0)),
            pl.BlockSpec((F, F2), lambda i: (0, 0)),
        ],
        out_specs=pl.BlockSpec((RB, F2), lambda i: (i, 0)),
    )(p, g, deg, b, W)


def _final_body(p_ref, g_ref, deg_ref, b_ref, o_ref):
    dinv = _dinv(deg_ref[...])
    z = (p_ref[0] + p_ref[1] + g_ref[...]) * dinv + b_ref[...]
    z2 = z[:, 0:2]
    m = jnp.max(z2, axis=1, keepdims=True)
    e = jnp.exp(z2 - m)
    lse = jnp.log(e[:, 0:1] + e[:, 1:2]) + m
    o_ref[...] = z2 - lse


def _final(p, g, deg, b):
    return pl.pallas_call(
        _final_body,
        out_shape=jax.ShapeDtypeStruct((N, 2), jnp.float32),
        grid=(GRID,),
        in_specs=[
            pl.BlockSpec((2, RB, 8), lambda i: (0, i, 0)),
            pl.BlockSpec((RB, 8), lambda i: (i, 0)),
            pl.BlockSpec((RB, 1), lambda i: (i, 0)),
            pl.BlockSpec((1, 8), lambda i: (0, 0)),
        ],
        out_specs=pl.BlockSpec((RB, 2), lambda i: (i, 0)),
    )(p, g, deg, b)


def kernel(x, edge_index, W1, b1, W2, b2, W3, b3):
    src2 = edge_index[0].reshape(NCHUNKS, CHUNK)
    dst2 = edge_index[1].reshape(NCHUNKS, CHUNK)

    deg = _make_deg()(dst2, jnp.zeros((NPAD,), jnp.float32))
    degc = deg.reshape(NPAD, 1)

    zeros16 = jnp.zeros((NPAD, 16), jnp.float32)
    zeros32 = jnp.zeros((NPAD, 32), jnp.float32)
    zeros8 = jnp.zeros((NPAD, 8), jnp.float32)

    g1 = _stage1(x, W1, degc)
    p1 = _make_agg(16)(src2, dst2, g1, zeros16)
    g2 = _mid(p1, g1, degc, b1.reshape(1, -1), W2)
    p2 = _make_agg(32)(src2, dst2, g2, zeros32)
    W3p = jnp.pad(W3, ((0, 0), (0, 6)))
    b3p = jnp.pad(b3, (0, 6)).reshape(1, 8)
    g3 = _mid(p2, g2, degc, b2.reshape(1, -1), W3p)
    p3 = _make_agg(8)(src2, dst2, g3, zeros8)
    return _final(p3, g3, degc, b3p)


# speedup vs baseline: 60.9530x; 1.1794x over previous
import functools

import jax
import jax.numpy as jnp
from jax import lax
from jax.experimental import pallas as pl
from jax.experimental.pallas import tpu as pltpu
from jax.experimental.pallas import tpu_sc as plsc

N = 10000
E = 320000
NC, NS, L = 2, 16, 16
NW = NC * NS

NPAD = 10240
ROWS_PER_TILE = NPAD // NS

CHUNK = 128
NCHUNKS = E // CHUNK
AGG_Q, AGG_R = NCHUNKS // NW, NCHUNKS % NW
AGG_MAX = AGG_Q + 1
DEG_Q, DEG_R = NCHUNKS // NS, NCHUNKS % NS
DEG_MAX = DEG_Q + 1


def _sc_mesh():
    return plsc.VectorSubcoreMesh(
        core_axis_name="c", subcore_axis_name="s",
        num_cores=NC, num_subcores=NS)


@functools.cache
def _make_deg():
    return functools.partial(
        pl.kernel,
        out_type=jax.ShapeDtypeStruct((NPAD,), jnp.float32),
        mesh=_sc_mesh(),
        compiler_params=pltpu.CompilerParams(use_tc_tiling_on_sc=False),
        scratch_types=[
            pltpu.VMEM((DEG_MAX, CHUNK), jnp.int32),
            pltpu.VMEM((CHUNK,), jnp.float32),
            pltpu.VMEM_SHARED((NPAD,), jnp.float32),
            pltpu.SemaphoreType.DMA,
            pltpu.SemaphoreType.DMA,
            pltpu.SemaphoreType.DMA,
            pltpu.SemaphoreType.DMA,
        ],
    )(_deg_body)


def _deg_body(dst_hbm, zeros_hbm, deg_out, dstv, onesv, acc, s0, s1, s2, s3):
    c = lax.axis_index("c")
    s = lax.axis_index("s")
    ssem = (s0, s1, s2, s3)
    nch = jnp.where(s >= NS - DEG_R, DEG_Q + 1, DEG_Q)
    base = DEG_Q * s + jnp.maximum(s - (NS - DEG_R), 0)
    one = jnp.full((L,), 1.0, dtype=jnp.float32)
    for i in range(CHUNK // L):
        onesv[pl.ds(i * L, L)] = one
    pltpu.sync_copy(dst_hbm.at[pl.ds(NCHUNKS + base, DEG_MAX)], dstv)
    stripe = pl.ds(s * (NPAD // NS), NPAD // NS)
    pltpu.sync_copy(zeros_hbm.at[stripe], acc.at[stripe])
    plsc.subcore_barrier()

    def s_fire(j, slot):
        pltpu.async_copy(onesv, acc.at[dstv.at[j]], ssem[slot], add=True)

    def s_wait(j, slot):
        pltpu.make_async_copy(onesv, acc.at[dstv.at[j]], ssem[slot]).wait()

    def grp(k, carry):
        for b in range(4):
            j = 4 * k + b

            @pl.when((j >= 4) & (j - 4 < nch))
            def _():
                s_wait(j - 4, b)

            @pl.when(j < nch)
            def _():
                s_fire(j, b)
        return carry

    lax.fori_loop(0, (DEG_MAX + 4 + 3) // 4, grp, 0)
    plsc.subcore_barrier()

    @pl.when(c == 0)
    def _():
        pltpu.sync_copy(acc.at[stripe], deg_out.at[stripe])


@functools.cache
def _make_agg(F):
    @functools.partial(
        pl.kernel,
        out_type=jax.ShapeDtypeStruct((NC, NPAD, F), jnp.float32),
        mesh=_sc_mesh(),
        compiler_params=pltpu.CompilerParams(use_tc_tiling_on_sc=False),
        scratch_types=[
            pltpu.VMEM((AGG_MAX, CHUNK), jnp.int32),
            pltpu.VMEM((AGG_MAX, CHUNK), jnp.int32),
            pltpu.VMEM((CHUNK, F), jnp.float32),
            pltpu.VMEM((CHUNK, F), jnp.float32),
            pltpu.VMEM((CHUNK, F), jnp.float32),
            pltpu.VMEM((CHUNK, F), jnp.float32),
            pltpu.VMEM_SHARED((NPAD, F), jnp.float32),
            pltpu.VMEM_SHARED((NPAD, F), jnp.float32),
            pltpu.SemaphoreType.DMA,
            pltpu.SemaphoreType.DMA,
            pltpu.SemaphoreType.DMA,
            pltpu.SemaphoreType.DMA,
            pltpu.SemaphoreType.DMA,
            pltpu.SemaphoreType.DMA,
            pltpu.SemaphoreType.DMA,
            pltpu.SemaphoreType.DMA,
        ],
    )
    def agg(edge_hbm, g_hbm, zeros_hbm, out_hbm,
            srcv, dstv, r0, r1, r2, r3, acc, gsh,
            g0, g1, g2, g3, s0, s1, s2, s3):
        c = lax.axis_index("c")
        s = lax.axis_index("s")
        w = c * NS + s
        rows = (r0, r1, r2, r3)
        gsem = (g0, g1, g2, g3)
        ssem = (s0, s1, s2, s3)
        nch = jnp.where(w >= NW - AGG_R, AGG_Q + 1, AGG_Q)
        base = AGG_Q * w + jnp.maximum(w - (NW - AGG_R), 0)
        pltpu.sync_copy(edge_hbm.at[pl.ds(base, AGG_MAX)], srcv)
        pltpu.sync_copy(edge_hbm.at[pl.ds(NCHUNKS + base, AGG_MAX)], dstv)
        stripe = pl.ds(s * ROWS_PER_TILE, ROWS_PER_TILE)
        pltpu.sync_copy(zeros_hbm.at[stripe], acc.at[stripe])
        pltpu.sync_copy(g_hbm.at[stripe], gsh.at[stripe])
        plsc.subcore_barrier()

        def g_issue(j, slot):
            pltpu.async_copy(gsh.at[srcv.at[j]], rows[slot], gsem[slot])

        def g_wait(j, slot):
            pltpu.make_async_copy(gsh.at[srcv.at[j]], rows[slot], gsem[slot]).wait()

        def s_fire(j, slot):
            pltpu.async_copy(rows[slot], acc.at[dstv.at[j]], ssem[slot], add=True)

        def s_wait(j, slot):
            pltpu.make_async_copy(rows[slot], acc.at[dstv.at[j]], ssem[slot]).wait()

        g_issue(0, 0)
        g_issue(1, 1)

        def grp(k, carry):
            for b in range(4):
                j = 4 * k + b
                s2 = (b + 2) % 4

                @pl.when((j >= 2) & (j - 2 < nch))
                def _():
                    s_wait(j - 2, s2)

                @pl.when(j + 2 < nch)
                def _():
                    g_issue(j + 2, s2)

                @pl.when(j < nch)
                def _():
                    g_wait(j, b)
                    s_fire(j, b)
            return carry

        lax.fori_loop(0, (AGG_MAX + 2 + 3) // 4, grp, 0)
        plsc.subcore_barrier()
        pltpu.sync_copy(acc.at[stripe], out_hbm.at[c].at[stripe])

    return agg


RB = 2048
GRID = NPAD // RB


def _dinv(deg_blk):
    return lax.rsqrt(deg_blk + 1.0)


def _mm_body(x_ref, w_ref, o_ref):
    o_ref[...] = jnp.dot(x_ref[...], w_ref[...],
                         preferred_element_type=jnp.float32)


def _mm(x, W1):
    return pl.pallas_call(
        _mm_body,
        out_shape=jax.ShapeDtypeStruct((NPAD, 16), jnp.float32),
        grid=(GRID,),
        in_specs=[
            pl.BlockSpec((RB, 128), lambda i: (i, 0)),
            pl.BlockSpec((128, 16), lambda i: (0, 0)),
        ],
        out_specs=pl.BlockSpec((RB, 16), lambda i: (i, 0)),
    )(x, W1)


def _scale_body(h_ref, deg_ref, o_ref):
    o_ref[...] = h_ref[...] * _dinv(deg_ref[...])


def _scale(h, deg):
    return pl.pallas_call(
        _scale_body,
        out_shape=jax.ShapeDtypeStruct((NPAD, 16), jnp.float32),
        grid=(GRID,),
        in_specs=[
            pl.BlockSpec((RB, 16), lambda i: (i, 0)),
            pl.BlockSpec((RB, 1), lambda i: (i, 0)),
        ],
        out_specs=pl.BlockSpec((RB, 16), lambda i: (i, 0)),
    )(h, deg)


def _mid_body(p_ref, g_ref, deg_ref, b_ref, w_ref, o_ref):
    dinv = _dinv(deg_ref[...])
    z = (p_ref[0] + p_ref[1] + g_ref[...]) * dinv + b_ref[...]
    h = jnp.maximum(z, 0.0)
    o_ref[...] = jnp.dot(h, w_ref[...], preferred_element_type=jnp.float32) * dinv


def _mid(p, g, deg, b, W):
    F = g.shape[1]
    F2 = W.shape[1]
    return pl.pallas_call(
        _mid_body,
        out_shape=jax.ShapeDtypeStruct((NPAD, F2), jnp.float32),
        grid=(GRID,),
        in_specs=[
            pl.BlockSpec((2, RB, F), lambda i: (0, i, 0)),
            pl.BlockSpec((RB, F), lambda i: (i, 0)),
            pl.BlockSpec((RB, 1), lambda i: (i, 0)),
            pl.BlockSpec((1, F), lambda i: (0, 0)),
            pl.BlockSpec((F, F2), lambda i: (0, 0)),
        ],
        out_specs=pl.BlockSpec((RB, F2), lambda i: (i, 0)),
    )(p, g, deg, b, W)


def _final_body(p_ref, g_ref, deg_ref, b_ref, o_ref):
    dinv = _dinv(deg_ref[...])
    z = (p_ref[0] + p_ref[1] + g_ref[...]) * dinv + b_ref[...]
    z2 = z[:, 0:2]
    m = jnp.max(z2, axis=1, keepdims=True)
    e = jnp.exp(z2 - m)
    lse = jnp.log(e[:, 0:1] + e[:, 1:2]) + m
    o_ref[...] = z2 - lse


def _final(p, g, deg, b):
    return pl.pallas_call(
        _final_body,
        out_shape=jax.ShapeDtypeStruct((NPAD, 2), jnp.float32),
        grid=(GRID,),
        in_specs=[
            pl.BlockSpec((2, RB, 8), lambda i: (0, i, 0)),
            pl.BlockSpec((RB, 8), lambda i: (i, 0)),
            pl.BlockSpec((RB, 1), lambda i: (i, 0)),
            pl.BlockSpec((1, 8), lambda i: (0, 0)),
        ],
        out_specs=pl.BlockSpec((RB, 2), lambda i: (i, 0)),
    )(p, g, deg, b)


def kernel(x, edge_index, W1, b1, W2, b2, W3, b3):
    ei = edge_index.reshape(2 * NCHUNKS, CHUNK)

    h1 = _mm(x, W1)
    deg = _make_deg()(ei, jnp.zeros((NPAD,), jnp.float32))
    degc = deg.reshape(NPAD, 1)

    zeros16 = jnp.zeros((NPAD, 16), jnp.float32)
    zeros32 = jnp.zeros((NPAD, 32), jnp.float32)
    zeros8 = jnp.zeros((NPAD, 8), jnp.float32)

    g1 = _scale(h1, degc)
    p1 = _make_agg(16)(ei, g1, zeros16)
    g2 = _mid(p1, g1, degc, b1.reshape(1, -1), W2)
    p2 = _make_agg(32)(ei, g2, zeros32)
    W3p = jnp.pad(W3, ((0, 0), (0, 6)))
    b3p = jnp.pad(b3, (0, 6)).reshape(1, 8)
    g3 = _mid(p2, g2, degc, b2.reshape(1, -1), W3p)
    p3 = _make_agg(8)(ei, g3, zeros8)
    return _final(p3, g3, degc, b3p)[:N]


# speedup vs baseline: 61.3707x; 1.0069x over previous
import functools

import jax
import jax.numpy as jnp
from jax import lax
from jax.experimental import pallas as pl
from jax.experimental.pallas import tpu as pltpu
from jax.experimental.pallas import tpu_sc as plsc

N = 10000
E = 320000
NC, NS, L = 2, 16, 16
NW = NC * NS

NPAD = 10240
ROWS_PER_TILE = NPAD // NS

CHUNK = 128
NCHUNKS = E // CHUNK
AGG_Q, AGG_R = NCHUNKS // NW, NCHUNKS % NW
AGG_MAX = AGG_Q + 1
DEG_Q, DEG_R = NCHUNKS // NS, NCHUNKS % NS
DEG_MAX = DEG_Q + 1


def _sc_mesh():
    return plsc.VectorSubcoreMesh(
        core_axis_name="c", subcore_axis_name="s",
        num_cores=NC, num_subcores=NS)


@functools.cache
def _make_deg():
    return functools.partial(
        pl.kernel,
        out_type=jax.ShapeDtypeStruct((NPAD,), jnp.float32),
        mesh=_sc_mesh(),
        compiler_params=pltpu.CompilerParams(use_tc_tiling_on_sc=False),
        scratch_types=[
            pltpu.VMEM((DEG_MAX, CHUNK), jnp.int32),
            pltpu.VMEM((CHUNK,), jnp.float32),
            pltpu.VMEM_SHARED((NPAD,), jnp.float32),
            pltpu.SemaphoreType.DMA,
            pltpu.SemaphoreType.DMA,
            pltpu.SemaphoreType.DMA,
            pltpu.SemaphoreType.DMA,
        ],
    )(_deg_body)


def _deg_body(dst_hbm, zeros_hbm, deg_out, dstv, onesv, acc, s0, s1, s2, s3):
    c = lax.axis_index("c")
    s = lax.axis_index("s")
    ssem = (s0, s1, s2, s3)
    nch = jnp.where(s >= NS - DEG_R, DEG_Q + 1, DEG_Q)
    base = DEG_Q * s + jnp.maximum(s - (NS - DEG_R), 0)
    one = jnp.full((L,), 1.0, dtype=jnp.float32)
    for i in range(CHUNK // L):
        onesv[pl.ds(i * L, L)] = one
    pltpu.sync_copy(dst_hbm.at[pl.ds(NCHUNKS + base, DEG_MAX)], dstv)
    stripe = pl.ds(s * (NPAD // NS), NPAD // NS)
    pltpu.sync_copy(zeros_hbm.at[stripe], acc.at[stripe])
    plsc.subcore_barrier()

    def s_fire(j, slot):
        pltpu.async_copy(onesv, acc.at[dstv.at[j]], ssem[slot], add=True)

    def s_wait(j, slot):
        pltpu.make_async_copy(onesv, acc.at[dstv.at[j]], ssem[slot]).wait()

    def grp(k, carry):
        for b in range(4):
            j = 4 * k + b

            @pl.when((j >= 4) & (j - 4 < nch))
            def _():
                s_wait(j - 4, b)

            @pl.when(j < nch)
            def _():
                s_fire(j, b)
        return carry

    lax.fori_loop(0, (DEG_MAX + 4 + 3) // 4, grp, 0)
    plsc.subcore_barrier()

    @pl.when(c == 0)
    def _():
        pltpu.sync_copy(acc.at[stripe], deg_out.at[stripe])


@functools.cache
def _make_agg(F):
    @functools.partial(
        pl.kernel,
        out_type=jax.ShapeDtypeStruct((NC, NPAD, F), jnp.float32),
        mesh=_sc_mesh(),
        compiler_params=pltpu.CompilerParams(use_tc_tiling_on_sc=False),
        scratch_types=[
            pltpu.VMEM((AGG_MAX, CHUNK), jnp.int32),
            pltpu.VMEM((AGG_MAX, CHUNK), jnp.int32),
            pltpu.VMEM((CHUNK, F), jnp.float32),
            pltpu.VMEM((CHUNK, F), jnp.float32),
            pltpu.VMEM((CHUNK, F), jnp.float32),
            pltpu.VMEM((CHUNK, F), jnp.float32),
            pltpu.VMEM_SHARED((NPAD, F), jnp.float32),
            pltpu.VMEM_SHARED((NPAD, F), jnp.float32),
            pltpu.SemaphoreType.DMA,
            pltpu.SemaphoreType.DMA,
            pltpu.SemaphoreType.DMA,
            pltpu.SemaphoreType.DMA,
            pltpu.SemaphoreType.DMA,
            pltpu.SemaphoreType.DMA,
            pltpu.SemaphoreType.DMA,
            pltpu.SemaphoreType.DMA,
        ],
    )
    def agg(edge_hbm, g_hbm, zeros_hbm, out_hbm,
            srcv, dstv, r0, r1, r2, r3, acc, gsh,
            g0, g1, g2, g3, s0, s1, s2, s3):
        c = lax.axis_index("c")
        s = lax.axis_index("s")
        w = c * NS + s
        rows = (r0, r1, r2, r3)
        gsem = (g0, g1, g2, g3)
        ssem = (s0, s1, s2, s3)
        nch = jnp.where(w >= NW - AGG_R, AGG_Q + 1, AGG_Q)
        base = AGG_Q * w + jnp.maximum(w - (NW - AGG_R), 0)
        pltpu.sync_copy(edge_hbm.at[pl.ds(base, AGG_MAX)], srcv)
        pltpu.sync_copy(edge_hbm.at[pl.ds(NCHUNKS + base, AGG_MAX)], dstv)
        stripe = pl.ds(s * ROWS_PER_TILE, ROWS_PER_TILE)
        pltpu.sync_copy(zeros_hbm.at[stripe], acc.at[stripe])
        pltpu.sync_copy(g_hbm.at[stripe], gsh.at[stripe])
        plsc.subcore_barrier()

        def g_issue(j, slot):
            pltpu.async_copy(gsh.at[srcv.at[j]], rows[slot], gsem[slot])

        def g_wait(j, slot):
            pltpu.make_async_copy(gsh.at[srcv.at[j]], rows[slot], gsem[slot]).wait()

        def s_fire(j, slot):
            pltpu.async_copy(rows[slot], acc.at[dstv.at[j]], ssem[slot], add=True)

        def s_wait(j, slot):
            pltpu.make_async_copy(rows[slot], acc.at[dstv.at[j]], ssem[slot]).wait()

        g_issue(0, 0)
        g_issue(1, 1)

        def grp(k, carry):
            for b in range(4):
                j = 4 * k + b
                s2 = (b + 2) % 4

                @pl.when((j >= 2) & (j - 2 < nch))
                def _():
                    s_wait(j - 2, s2)

                @pl.when(j + 2 < nch)
                def _():
                    g_issue(j + 2, s2)

                @pl.when(j < nch)
                def _():
                    g_wait(j, b)
                    s_fire(j, b)
            return carry

        lax.fori_loop(0, (AGG_MAX + 2 + 3) // 4, grp, 0)
        plsc.subcore_barrier()
        pltpu.sync_copy(acc.at[stripe], out_hbm.at[c].at[stripe])

    return agg


RB = 2048
GRID = NPAD // RB


def _dinv(deg_blk):
    return lax.rsqrt(deg_blk + 1.0)


def _stage1_body(x_ref, w_ref, deg_ref, o_ref):
    h = jnp.dot(x_ref[...], w_ref[...], preferred_element_type=jnp.float32)
    o_ref[...] = h * _dinv(deg_ref[...])


def _stage1(x, W1, deg):
    return pl.pallas_call(
        _stage1_body,
        out_shape=jax.ShapeDtypeStruct((NPAD, 16), jnp.float32),
        grid=(GRID,),
        in_specs=[
            pl.BlockSpec((RB, 128), lambda i: (i, 0)),
            pl.BlockSpec((128, 16), lambda i: (0, 0)),
            pl.BlockSpec((RB, 1), lambda i: (i, 0)),
        ],
        out_specs=pl.BlockSpec((RB, 16), lambda i: (i, 0)),
    )(x, W1, deg)


def _mid_body(p_ref, g_ref, deg_ref, b_ref, w_ref, o_ref):
    dinv = _dinv(deg_ref[...])
    z = (p_ref[0] + p_ref[1] + g_ref[...]) * dinv + b_ref[...]
    h = jnp.maximum(z, 0.0)
    o_ref[...] = jnp.dot(h, w_ref[...], preferred_element_type=jnp.float32) * dinv


def _mid(p, g, deg, b, W):
    F = g.shape[1]
    F2 = W.shape[1]
    return pl.pallas_call(
        _mid_body,
        out_shape=jax.ShapeDtypeStruct((NPAD, F2), jnp.float32),
        grid=(GRID,),
        in_specs=[
            pl.BlockSpec((2, RB, F), lambda i: (0, i, 0)),
            pl.BlockSpec((RB, F), lambda i: (i, 0)),
            pl.BlockSpec((RB, 1), lambda i: (i, 0)),
            pl.BlockSpec((1, F), lambda i: (0, 0)),
            pl.BlockSpec((F, F2), lambda i: (0, 0)),
        ],
        out_specs=pl.BlockSpec((RB, F2), lambda i: (i, 0)),
    )(p, g, deg, b, W)


def _final_body(p_ref, g_ref, deg_ref, b_ref, o_ref):
    dinv = _dinv(deg_ref[...])
    z = (p_ref[0] + p_ref[1] + g_ref[...]) * dinv + b_ref[...]
    z2 = z[:, 0:2]
    m = jnp.max(z2, axis=1, keepdims=True)
    e = jnp.exp(z2 - m)
    lse = jnp.log(e[:, 0:1] + e[:, 1:2]) + m
    o_ref[...] = z2 - lse


def _final(p, g, deg, b):
    return pl.pallas_call(
        _final_body,
        out_shape=jax.ShapeDtypeStruct((N, 2), jnp.float32),
        grid=(GRID,),
        in_specs=[
            pl.BlockSpec((2, RB, 8), lambda i: (0, i, 0)),
            pl.BlockSpec((RB, 8), lambda i: (i, 0)),
            pl.BlockSpec((RB, 1), lambda i: (i, 0)),
            pl.BlockSpec((1, 8), lambda i: (0, 0)),
        ],
        out_specs=pl.BlockSpec((RB, 2), lambda i: (i, 0)),
    )(p, g, deg, b)


def kernel(x, edge_index, W1, b1, W2, b2, W3, b3):
    ei = edge_index.reshape(2 * NCHUNKS, CHUNK)

    deg = _make_deg()(ei, jnp.zeros((NPAD,), jnp.float32))
    degc = deg.reshape(NPAD, 1)

    zeros16 = jnp.zeros((NPAD, 16), jnp.float32)
    zeros32 = jnp.zeros((NPAD, 32), jnp.float32)
    zeros8 = jnp.zeros((NPAD, 8), jnp.float32)

    g1 = _stage1(x, W1, degc)
    p1 = _make_agg(16)(ei, g1, zeros16)
    g2 = _mid(p1, g1, degc, b1.reshape(1, -1), W2)
    p2 = _make_agg(32)(ei, g2, zeros32)
    W3p = jnp.pad(W3, ((0, 0), (0, 6)))
    b3p = jnp.pad(b3, (0, 6)).reshape(1, 8)
    g3 = _mid(p2, g2, degc, b2.reshape(1, -1), W3p)
    p3 = _make_agg(8)(ei, g3, zeros8)
    return _final(p3, g3, degc, b3p)
